# baseline (speedup 1.0000x reference)
import functools

import jax
import jax.numpy as jnp
from jax import lax
from jax.experimental import pallas as pl
from jax.experimental.pallas import tpu as pltpu
from jax.experimental.pallas import tpu_sc as plsc

T_ROWS = 1000
D_IN = 128
D_OUT = 512
HALF = D_OUT // 2
B = 16384

_info = plsc.get_sparse_core_info()
NC, NS, L = _info.num_cores, _info.num_subcores, _info.num_lanes
NW = NC * NS
BPW = B // NW
C = 64
NCHUNK = BPW // C
NGBUF = 3
NSBUF = 2


def _table_body(pe_ref, w1_ref, b1_ref, w2_ref, b2_ref, out_ref):
    h = jnp.dot(pe_ref[...], w1_ref[...], preferred_element_type=jnp.float32)
    h = h + b1_ref[...]
    h = h * jax.nn.sigmoid(h)
    t = jnp.dot(h, w2_ref[...], preferred_element_type=jnp.float32) + b2_ref[...]
    u = jax.lax.bitcast_convert_type(t.astype(jnp.bfloat16), jnp.uint16)
    u = u.astype(jnp.uint32)
    packed = (u[:, HALF:] << 16) | u[:, :HALF]
    out_ref[...] = jax.lax.bitcast_convert_type(packed, jnp.int32)


def _compute_table(pe, W1, b1, W2, b2):
    return pl.pallas_call(
        _table_body,
        out_shape=jax.ShapeDtypeStruct((T_ROWS, HALF), jnp.int32),
    )(pe, W1, b1.reshape(1, D_OUT), W2, b2.reshape(1, D_OUT))


_mesh = plsc.VectorSubcoreMesh(core_axis_name="c", subcore_axis_name="s")

_NVEC = HALF // L


@functools.partial(
    pl.kernel,
    mesh=_mesh,
    compiler_params=pltpu.CompilerParams(needs_layout_passes=False),
    out_type=jax.ShapeDtypeStruct((B * D_OUT,), jnp.float32),
    scratch_types=[
        pltpu.VMEM((NCHUNK, C), jnp.int32),
        *[pltpu.VMEM((C, HALF), jnp.int32) for _ in range(NGBUF)],
        *[pltpu.VMEM((C * D_OUT,), jnp.float32) for _ in range(NSBUF)],
        *[pltpu.SemaphoreType.DMA for _ in range(NGBUF + NSBUF)],
    ],
)
def _sc_gather(table_hbm, idx_hbm, out_hbm, idx_v, *rest):
    gbufs = rest[:NGBUF]
    fbufs = rest[NGBUF:NGBUF + NSBUF]
    gsems = rest[NGBUF + NSBUF:NGBUF + NSBUF + NGBUF]
    ssems = rest[NGBUF + NSBUF + NGBUF:]
    wid = lax.axis_index("s") * NC + lax.axis_index("c")
    base = wid * BPW
    pltpu.sync_copy(idx_hbm.at[wid], idx_v)

    lane = lax.iota(jnp.int32, L)

    def unpack_chunk(src, dst):
        def row(r, carry):
            dbase = r * D_OUT
            rvec = jnp.full((L,), r, jnp.int32)
            for q in range(_NVEC):
                v = plsc.load_gather(src, [rvec, lane + (q * L)])
                lo = jax.lax.bitcast_convert_type(v << 16, jnp.float32)
                hi = jax.lax.bitcast_convert_type(v & jnp.int32(-65536), jnp.float32)
                dst[pl.ds(dbase + q * L, L)] = lo
                dst[pl.ds(dbase + HALF + q * L, L)] = hi
            return carry

        lax.fori_loop(0, C, row, 0)

    g = [None] * NCHUNK
    s = [None] * NCHUNK
    for c in range(min(NGBUF, NCHUNK)):
        g[c] = pltpu.async_copy(
            table_hbm.at[idx_v.at[c]], gbufs[c % NGBUF], gsems[c % NGBUF]
        )
    for c in range(NCHUNK):
        g[c].wait()
        if c - NSBUF >= 0:
            s[c - NSBUF].wait()
        unpack_chunk(gbufs[c % NGBUF], fbufs[c % NSBUF])
        s[c] = pltpu.async_copy(
            fbufs[c % NSBUF],
            out_hbm.at[pl.ds((base + c * C) * D_OUT, C * D_OUT)],
            ssems[c % NSBUF],
        )
        n = c + NGBUF
        if n < NCHUNK:
            g[n] = pltpu.async_copy(
                table_hbm.at[idx_v.at[n]], gbufs[n % NGBUF], gsems[n % NGBUF]
            )
    for c in range(max(0, NCHUNK - NSBUF), NCHUNK):
        s[c].wait()


def kernel(x, pe, W1, b1, W2, b2):
    table = _compute_table(pe, W1, b1, W2, b2)
    idx = x.astype(jnp.int32).reshape(NW, NCHUNK, C)
    return _sc_gather(table, idx).reshape(B, D_OUT)


# speedup vs baseline: 1.9321x; 1.9321x over previous
import functools

import jax
import jax.numpy as jnp
from jax import lax
from jax.experimental import pallas as pl
from jax.experimental.pallas import tpu as pltpu
from jax.experimental.pallas import tpu_sc as plsc

T_ROWS = 1000
T_PAD = 1024
D_IN = 128
D_OUT = 512
B = 16384
NSC = 8192
NT = B - NSC
TB = 1024
NTB = NT // TB

_info = plsc.get_sparse_core_info()
NC, NS = _info.num_cores, _info.num_subcores
NW = NC * NS
BPW = NSC // NW
C = 64
NCHUNK = BPW // C
NBUF = 3


def _table_body(pe_ref, w1_ref, b1_ref, w2_ref, b2_ref, out_ref, outb_ref):
    h = jnp.dot(pe_ref[...], w1_ref[...], preferred_element_type=jnp.float32)
    h = h + b1_ref[...]
    h = h * jax.nn.sigmoid(h)
    t = jnp.dot(h, w2_ref[...], preferred_element_type=jnp.float32) + b2_ref[...]
    out_ref[...] = t
    outb_ref[...] = t.astype(jnp.bfloat16)


def _compute_table(pe, W1, b1, W2, b2):
    return pl.pallas_call(
        _table_body,
        out_shape=(
            jax.ShapeDtypeStruct((T_PAD, D_OUT), jnp.float32),
            jax.ShapeDtypeStruct((T_PAD, D_OUT), jnp.bfloat16),
        ),
    )(pe, W1, b1.reshape(1, D_OUT), W2, b2.reshape(1, D_OUT))


def _tail_body(xb_ref, tbl_ref, out_ref):
    xb = xb_ref[0, 0, :]
    cols = lax.broadcasted_iota(jnp.int32, (TB, T_PAD), 1)
    oh = (xb[:, None] == cols).astype(jnp.bfloat16)
    out_ref[...] = jnp.dot(oh, tbl_ref[...], preferred_element_type=jnp.float32)


def _compute_tail(x_tail, table_bf16):
    x3 = x_tail.reshape(NTB, 1, TB)
    return pl.pallas_call(
        _tail_body,
        grid=(NTB,),
        in_specs=[
            pl.BlockSpec((1, 1, TB), lambda i: (i, 0, 0)),
            pl.BlockSpec((T_PAD, D_OUT), lambda i: (0, 0)),
        ],
        out_specs=pl.BlockSpec((TB, D_OUT), lambda i: (i, 0)),
        out_shape=jax.ShapeDtypeStruct((NT, D_OUT), jnp.float32),
    )(x3, table_bf16)


_mesh = plsc.VectorSubcoreMesh(core_axis_name="c", subcore_axis_name="s")


@functools.partial(
    pl.kernel,
    mesh=_mesh,
    out_type=jax.ShapeDtypeStruct((B, D_OUT), jnp.float32),
    scratch_types=[
        pltpu.VMEM((NCHUNK, C), jnp.int32),
        *[pltpu.VMEM((C, D_OUT), jnp.float32) for _ in range(NBUF)],
        *[pltpu.SemaphoreType.DMA for _ in range(2 * NBUF)],
    ],
)
def _sc_gather(table_hbm, idx_hbm, out_hbm, idx_v, *rest):
    bufs = rest[:NBUF]
    gsems = rest[NBUF:2 * NBUF]
    ssems = rest[2 * NBUF:]
    wid = lax.axis_index("s") * NC + lax.axis_index("c")
    base = wid * BPW
    pltpu.sync_copy(idx_hbm.at[wid], idx_v)
    la = NBUF - 1
    g = [None] * NCHUNK
    s = [None] * NCHUNK
    for c in range(min(la, NCHUNK)):
        g[c] = pltpu.async_copy(
            table_hbm.at[idx_v.at[c]], bufs[c % NBUF], gsems[c % NBUF]
        )
    for c in range(NCHUNK):
        g[c].wait()
        n = c + la
        if n < NCHUNK:
            if n - NBUF >= 0:
                s[n - NBUF].wait()
            g[n] = pltpu.async_copy(
                table_hbm.at[idx_v.at[n]], bufs[n % NBUF], gsems[n % NBUF]
            )
        s[c] = pltpu.async_copy(
            bufs[c % NBUF], out_hbm.at[pl.ds(base + c * C, C)], ssems[c % NBUF]
        )
    for c in range(max(0, NCHUNK - NBUF), NCHUNK):
        s[c].wait()


def kernel(x, pe, W1, b1, W2, b2):
    pe_pad = jnp.pad(pe, ((0, T_PAD - T_ROWS), (0, 0)))
    table, table_bf16 = _compute_table(pe_pad, W1, b1, W2, b2)
    xi = x.astype(jnp.int32)
    idx = xi[:NSC].reshape(NW, NCHUNK, C)
    out = _sc_gather(table, idx)
    tail = _compute_tail(xi[NSC:], table_bf16)
    return lax.dynamic_update_slice(out, tail, (NSC, 0))


# speedup vs baseline: 2.2096x; 1.1436x over previous
import functools

import jax
import jax.numpy as jnp
from jax import lax
from jax.experimental import pallas as pl
from jax.experimental.pallas import tpu as pltpu
from jax.experimental.pallas import tpu_sc as plsc

T_ROWS = 1000
D_IN = 128
D_OUT = 512
B = 16384

_info = plsc.get_sparse_core_info()
NC, NS = _info.num_cores, _info.num_subcores
NW = NC * NS
BPW = B // NW
C = 64
NCHUNK = BPW // C
NBUF = 3


def _table_body(pe_ref, w1_ref, b1_ref, w2_ref, b2_ref, out_ref):
    h = jnp.dot(pe_ref[...], w1_ref[...], preferred_element_type=jnp.float32)
    h = h + b1_ref[...]
    h = h * jax.nn.sigmoid(h)
    out_ref[...] = (
        jnp.dot(h, w2_ref[...], preferred_element_type=jnp.float32) + b2_ref[...]
    )


def _compute_table(pe, W1, b1, W2, b2):
    return pl.pallas_call(
        _table_body,
        out_shape=jax.ShapeDtypeStruct((T_ROWS, D_OUT), jnp.float32),
    )(pe, W1, b1.reshape(1, D_OUT), W2, b2.reshape(1, D_OUT))


_mesh = plsc.VectorSubcoreMesh(core_axis_name="c", subcore_axis_name="s")


@functools.partial(
    pl.kernel,
    mesh=_mesh,
    out_type=jax.ShapeDtypeStruct((B, D_OUT), jnp.float32),
    scratch_types=[
        pltpu.VMEM((BPW,), jnp.int32),
        *[pltpu.VMEM((C, D_OUT), jnp.float32) for _ in range(NBUF)],
        *[pltpu.SemaphoreType.DMA for _ in range(2 * NBUF)],
    ],
)
def _sc_gather(table_hbm, idx_hbm, out_hbm, idx_v, *rest):
    bufs = rest[:NBUF]
    gsems = rest[NBUF:2 * NBUF]
    ssems = rest[2 * NBUF:]
    wid = lax.axis_index("s") * NC + lax.axis_index("c")
    base = wid * BPW
    pltpu.sync_copy(idx_hbm.at[pl.ds(base, BPW)], idx_v)
    la = NBUF - 1
    g = [None] * NCHUNK
    s = [None] * NCHUNK
    for c in range(min(la, NCHUNK)):
        g[c] = pltpu.async_copy(
            table_hbm.at[idx_v.at[pl.ds(c * C, C)]], bufs[c % NBUF], gsems[c % NBUF]
        )
    for c in range(NCHUNK):
        g[c].wait()
        n = c + la
        if n < NCHUNK:
            if n - NBUF >= 0:
                s[n - NBUF].wait()
            g[n] = pltpu.async_copy(
                table_hbm.at[idx_v.at[pl.ds(n * C, C)]],
                bufs[n % NBUF],
                gsems[n % NBUF],
            )
        s[c] = pltpu.async_copy(
            bufs[c % NBUF], out_hbm.at[pl.ds(base + c * C, C)], ssems[c % NBUF]
        )
    for c in range(max(0, NCHUNK - NBUF), NCHUNK):
        s[c].wait()


def kernel(x, pe, W1, b1, W2, b2):
    table = _compute_table(pe, W1, b1, W2, b2)
    return _sc_gather(table, x.astype(jnp.int32))
